# baseline (speedup 1.0000x reference)
import functools

import jax
import jax.numpy as jnp
from jax import lax
from jax.experimental import pallas as pl
from jax.experimental.pallas import tpu as pltpu, tpu_sc as plsc

NC = 2
NS = 16
NW = NC * NS
L = 16
CH = 128
GROUP = 16


def _round_up(a, b):
    return -(-a // b) * b


def _softplus(z):
    return jnp.maximum(z, 0.0) + jnp.log(1.0 + jnp.exp(-jnp.abs(z)))


def _dot_like_ref(a, b):
    return jnp.dot(a.astype(jnp.bfloat16), b.astype(jnp.bfloat16),
                   preferred_element_type=jnp.float32)


def _tc_proj_kernel(x_ref, w_ref, p_ref, root_ref):
    y = _dot_like_ref(x_ref[...], w_ref[...])
    col = lax.broadcasted_iota(jnp.int32, p_ref.shape, 1)
    p_ref[...] = jnp.where(col == (L - 1), 1.0, y[:, :L])
    root_ref[...] = y[:, L:2 * L]


def _tc_mid_kernel(acc_ref, root_ref, b_ref, w_ref, p_ref, root2_ref):
    n = root_ref.shape[0]
    a = acc_ref[0, :n, :] + acc_ref[1, :n, :]
    cnt = a[:, L - 1:L]
    z = a / jnp.maximum(cnt, 1.0) + root_ref[...] + b_ref[...]
    h = _softplus(z)
    y = _dot_like_ref(h, w_ref[...])
    col = lax.broadcasted_iota(jnp.int32, p_ref.shape, 1)
    p_ref[...] = jnp.where(col == (L - 1), 1.0, y[:, :L])
    root2_ref[...] = y[:, L:2 * L]


def _tc_final_kernel(acc_ref, root_ref, b_ref, wfc_ref, bfc_ref, out_ref):
    n = root_ref.shape[0]
    a = acc_ref[0, :n, :] + acc_ref[1, :n, :]
    cnt = a[:, L - 1:L]
    z = a / jnp.maximum(cnt, 1.0) + root_ref[...] + b_ref[...]
    h = _softplus(z)
    hb = h.astype(jnp.bfloat16).astype(jnp.float32)
    wb = wfc_ref[...].astype(jnp.bfloat16).astype(jnp.float32)
    pre = jnp.sum(hb * wb, axis=1, keepdims=True) + bfc_ref[0, 0]
    out_ref[...] = pre - jnp.min(pre)


def _make_seg_sum(n_acc, j_chunks, rps):
    mesh = plsc.VectorSubcoreMesh(core_axis_name="c", subcore_axis_name="s")

    @functools.partial(
        pl.kernel,
        out_type=jax.ShapeDtypeStruct((NC, n_acc, L), jnp.float32),
        mesh=mesh,
        scratch_types=[
            pltpu.VMEM((j_chunks, CH), jnp.int32),
            pltpu.VMEM((j_chunks, CH), jnp.int32),
            pltpu.VMEM((2, GROUP, CH, L), jnp.float32),
            pltpu.VMEM((rps, L), jnp.float32),
            pltpu.VMEM_SHARED((n_acc, L), jnp.float32),
            pltpu.SemaphoreType.DMA,
            pltpu.SemaphoreType.DMA,
            pltpu.SemaphoreType.DMA,
            pltpu.SemaphoreType.DMA,
        ],
        compiler_params=pltpu.CompilerParams(
            use_tc_tiling_on_sc=False, disable_bounds_checks=True),
    )
    def seg_sum(src_hbm, dst_hbm, table_hbm, out_hbm,
                src_v, dst_v, rows_v, zero_v, acc_sh,
                gsem0, gsem1, ssem0, ssem1):
        cid = lax.axis_index("c")
        sid = lax.axis_index("s")
        wid = cid * NS + sid
        gsems = (gsem0, gsem1)
        ssems = (ssem0, ssem1)

        def zero_body(i, c):
            zero_v[i, :] = jnp.zeros((L,), jnp.float32)
            return c

        lax.fori_loop(0, rps, zero_body, 0)
        pltpu.sync_copy(zero_v, acc_sh.at[pl.ds(sid * rps, rps)])
        plsc.subcore_barrier()

        pltpu.sync_copy(src_hbm.at[wid], src_v)
        pltpu.sync_copy(dst_hbm.at[wid], dst_v)

        ngroups = j_chunks // GROUP
        gdescs = [None] * ngroups
        sdescs = [None] * ngroups
        for g in range(ngroups + 1):
            if g < ngroups:
                if g >= 2:
                    for d in sdescs[g - 2]:
                        d.wait()
                buf = rows_v.at[g % 2]
                gdescs[g] = [
                    pltpu.async_copy(
                        table_hbm.at[src_v.at[g * GROUP + b]],
                        buf.at[b], gsems[g % 2])
                    for b in range(GROUP)]
            if g >= 1:
                p = g - 1
                for d in gdescs[p]:
                    d.wait()
                buf = rows_v.at[p % 2]
                sdescs[p] = [
                    pltpu.async_copy(
                        buf.at[b], acc_sh.at[dst_v.at[p * GROUP + b]],
                        ssems[p % 2], add=True)
                    for b in range(GROUP)]
        for d in sdescs[ngroups - 2]:
            d.wait()
        for d in sdescs[ngroups - 1]:
            d.wait()

        plsc.subcore_barrier()
        pltpu.sync_copy(acc_sh.at[pl.ds(sid * rps, rps)],
                        out_hbm.at[cid].at[pl.ds(sid * rps, rps)])

    return seg_sum


def kernel(x, edge_index, W_rel1, W_root1, b1, W_rel2, W_root2, b2,
           W_fc, b_fc):
    N, D = x.shape
    E = edge_index.shape[1]
    R1 = W_rel1.shape[1]
    R2 = W_rel2.shape[1]

    j_chunks = _round_up(-(-E // (NW * CH)), GROUP)
    e_pad = NW * j_chunks * CH
    rps = _round_up(N + 1 + 1024, NS * 8) // NS
    n_acc = rps * NS

    src = edge_index[0].astype(jnp.int32)
    dst = edge_index[1].astype(jnp.int32)
    pad_iota = jnp.arange(e_pad - E, dtype=jnp.int32)
    pad_src = pad_iota % N
    pad_dst = N + pad_iota % (n_acc - N)
    src = jnp.concatenate([src, pad_src]).reshape(NW, j_chunks, CH)
    dst = jnp.concatenate([dst, pad_dst]).reshape(NW, j_chunks, CH)

    w1 = jnp.zeros((D, 2 * L), jnp.float32)
    w1 = w1.at[:, :R1].set(W_rel1).at[:, L:L + R1].set(W_root1)
    b1p = jnp.zeros((1, L), jnp.float32).at[0, :R1].set(b1)
    w2 = jnp.zeros((L, 2 * L), jnp.float32)
    w2 = w2.at[:R1, :R2].set(W_rel2).at[:R1, L:L + R2].set(W_root2)
    b2p = jnp.zeros((1, L), jnp.float32).at[0, :R2].set(b2)
    wfcp = jnp.zeros((1, L), jnp.float32).at[0, :R2].set(W_fc[:, 0])
    bfcp = b_fc.reshape(1, 1)

    p1, xroot1 = pl.pallas_call(
        _tc_proj_kernel,
        out_shape=[jax.ShapeDtypeStruct((N, L), jnp.float32),
                   jax.ShapeDtypeStruct((N, L), jnp.float32)],
    )(x, w1)

    seg_sum = _make_seg_sum(n_acc, j_chunks, rps)
    acc1 = seg_sum(src, dst, p1)

    p2, hroot2 = pl.pallas_call(
        _tc_mid_kernel,
        out_shape=[jax.ShapeDtypeStruct((N, L), jnp.float32),
                   jax.ShapeDtypeStruct((N, L), jnp.float32)],
    )(acc1, xroot1, b1p, w2)

    acc2 = seg_sum(src, dst, p2)

    out = pl.pallas_call(
        _tc_final_kernel,
        out_shape=jax.ShapeDtypeStruct((N, 1), jnp.float32),
    )(acc2, hroot2, b2p, wfcp, bfcp)

    return out


# speedup vs baseline: 1.4031x; 1.4031x over previous
import functools

import jax
import jax.numpy as jnp
import numpy as np
from jax import lax
from jax.experimental import pallas as pl
from jax.experimental.pallas import tpu as pltpu, tpu_sc as plsc

NC = 2
NS = 16
NW = NC * NS
L = 16
G = 8
CH = 128
GROUP = 16


def _round_up(a, b):
    return -(-a // b) * b


def _softplus(z):
    return jnp.maximum(z, 0.0) + jnp.log(1.0 + jnp.exp(-jnp.abs(z)))


def _dot_like_ref(a, b):
    return jnp.dot(a.astype(jnp.bfloat16), b.astype(jnp.bfloat16),
                   preferred_element_type=jnp.float32)


_E15 = np.zeros((L, L), np.float32)
_E15[L - 1, :] = 1.0
_CNT_BCAST = np.kron(np.eye(G, dtype=np.float32), _E15)
_GSUM = np.zeros((G * L, G), np.float32)
for _g in range(G):
    _GSUM[_g * L:_g * L + L, _g] = 1.0


def _tc_proj_kernel(n_pad, x_ref, w_ref, p_ref, root_ref):
    y = _dot_like_ref(x_ref[...], w_ref[...])
    zpad = jnp.zeros((n_pad, 2 * L), jnp.float32)
    y = jnp.concatenate([y, zpad], axis=0)
    r = p_ref.shape[0]
    blocks_p = [y[u * r:(u + 1) * r, :L] for u in range(G)]
    blocks_r = [y[u * r:(u + 1) * r, L:2 * L] for u in range(G)]
    p = jnp.concatenate(blocks_p, axis=1)
    col = lax.broadcasted_iota(jnp.int32, p.shape, 1)
    p_ref[...] = jnp.where(col % L == L - 1, 1.0, p)
    root_ref[...] = jnp.concatenate(blocks_r, axis=1)


def _tc_mid_kernel(acc_ref, root_ref, b_ref, wa_ref, wb_ref, cnt_ref,
                   p_ref, root2_ref):
    a = acc_ref[0] + acc_ref[1]
    cntb = jnp.dot(a, cnt_ref[...], preferred_element_type=jnp.float32)
    z = a / jnp.maximum(cntb, 1.0) + root_ref[...] + b_ref[...]
    h = _softplus(z)
    y = _dot_like_ref(h, wa_ref[...])
    col = lax.broadcasted_iota(jnp.int32, y.shape, 1)
    p_ref[...] = jnp.where(col % L == L - 1, 1.0, y)
    root2_ref[...] = _dot_like_ref(h, wb_ref[...])


def _tc_final_kernel(n, acc_ref, root_ref, b_ref, wfc_ref, bfc_ref,
                     cnt_ref, gsum_ref, out_ref):
    a = acc_ref[0] + acc_ref[1]
    cntb = jnp.dot(a, cnt_ref[...], preferred_element_type=jnp.float32)
    z = a / jnp.maximum(cntb, 1.0) + root_ref[...] + b_ref[...]
    h = _softplus(z)
    hw = (h.astype(jnp.bfloat16).astype(jnp.float32)
          * wfc_ref[...].astype(jnp.bfloat16).astype(jnp.float32))
    pre = jnp.dot(hw, gsum_ref[...],
                  preferred_element_type=jnp.float32) + bfc_ref[0, 0]
    r = out_ref.shape[0]
    node = (lax.broadcasted_iota(jnp.int32, pre.shape, 1) * r
            + lax.broadcasted_iota(jnp.int32, pre.shape, 0))
    m = jnp.min(jnp.where(node < n, pre, jnp.float32(3.0e38)))
    out_ref[...] = pre - m


def _make_seg_sum(n_acc, j_chunks, rps):
    mesh = plsc.VectorSubcoreMesh(core_axis_name="c", subcore_axis_name="s")

    @functools.partial(
        pl.kernel,
        out_type=jax.ShapeDtypeStruct((NC, n_acc, L), jnp.float32),
        mesh=mesh,
        scratch_types=[
            pltpu.VMEM((j_chunks, CH), jnp.int32),
            pltpu.VMEM((j_chunks, CH), jnp.int32),
            pltpu.VMEM((2, GROUP, CH, L), jnp.float32),
            pltpu.VMEM((rps, L), jnp.float32),
            pltpu.VMEM_SHARED((n_acc, L), jnp.float32),
            pltpu.SemaphoreType.DMA,
            pltpu.SemaphoreType.DMA,
            pltpu.SemaphoreType.DMA,
            pltpu.SemaphoreType.DMA,
        ],
        compiler_params=pltpu.CompilerParams(
            use_tc_tiling_on_sc=False, disable_bounds_checks=True),
    )
    def seg_sum(ei_hbm, table_hbm, out_hbm,
                src_v, dst_v, rows_v, zero_v, acc_sh,
                gsem0, gsem1, ssem0, ssem1):
        cid = lax.axis_index("c")
        sid = lax.axis_index("s")
        wid = cid * NS + sid
        gsems = (gsem0, gsem1)
        ssems = (ssem0, ssem1)

        def zero_body(i, c):
            zero_v[i, :] = jnp.zeros((L,), jnp.float32)
            return c

        lax.fori_loop(0, rps, zero_body, 0)
        pltpu.sync_copy(zero_v, acc_sh.at[pl.ds(sid * rps, rps)])
        plsc.subcore_barrier()

        pltpu.sync_copy(ei_hbm.at[wid], src_v)
        pltpu.sync_copy(ei_hbm.at[NW + wid], dst_v)

        ngroups = j_chunks // GROUP
        gdescs = [None] * ngroups
        sdescs = [None] * ngroups
        for g in range(ngroups + 1):
            if g < ngroups:
                if g >= 2:
                    for d in sdescs[g - 2]:
                        d.wait()
                buf = rows_v.at[g % 2]
                gdescs[g] = [
                    pltpu.async_copy(
                        table_hbm.at[src_v.at[g * GROUP + b]],
                        buf.at[b], gsems[g % 2])
                    for b in range(GROUP)]
            if g >= 1:
                p = g - 1
                for d in gdescs[p]:
                    d.wait()
                buf = rows_v.at[p % 2]
                sdescs[p] = [
                    pltpu.async_copy(
                        buf.at[b], acc_sh.at[dst_v.at[p * GROUP + b]],
                        ssems[p % 2], add=True)
                    for b in range(GROUP)]
        for d in sdescs[ngroups - 2]:
            d.wait()
        for d in sdescs[ngroups - 1]:
            d.wait()

        plsc.subcore_barrier()
        pltpu.sync_copy(acc_sh.at[pl.ds(sid * rps, rps)],
                        out_hbm.at[cid].at[pl.ds(sid * rps, rps)])

    return seg_sum


def kernel(x, edge_index, W_rel1, W_root1, b1, W_rel2, W_root2, b2,
           W_fc, b_fc):
    N, D = x.shape
    E = edge_index.shape[1]
    R1 = W_rel1.shape[1]
    R2 = W_rel2.shape[1]

    j_chunks = _round_up(-(-E // (NW * CH)), GROUP)
    e_pad = NW * j_chunks * CH
    R = _round_up(-(-(N + 1024) // G), 2 * NS)
    n_acc = R * G
    rps = n_acc // NS

    ei = edge_index.astype(jnp.int32)
    eip = (ei % R) * G + ei // R
    rng = np.arange(e_pad - E)
    pad_m = N + rng % (n_acc - N)
    pad_np = np.stack([rng % n_acc,
                       (pad_m % R) * G + pad_m // R]).astype(np.int32)
    eip = jnp.concatenate([eip, jnp.asarray(pad_np)], axis=1)
    eip = eip.reshape(2 * NW, j_chunks, CH)

    w1 = jnp.zeros((D, 2 * L), jnp.float32)
    w1 = w1.at[:, :R1].set(W_rel1).at[:, L:L + R1].set(W_root1)
    eye8 = jnp.eye(G, dtype=jnp.float32)
    w2a = jnp.zeros((L, L), jnp.float32).at[:R1, :R2].set(W_rel2)
    w2b = jnp.zeros((L, L), jnp.float32).at[:R1, :R2].set(W_root2)
    w2a = jnp.einsum("ab,cd->acbd", eye8, w2a).reshape(G * L, G * L)
    w2b = jnp.einsum("ab,cd->acbd", eye8, w2b).reshape(G * L, G * L)
    b1row = jnp.tile(jnp.zeros((L,), jnp.float32).at[:R1].set(b1),
                     (G,)).reshape(1, G * L)
    b2row = jnp.tile(jnp.zeros((L,), jnp.float32).at[:R2].set(b2),
                     (G,)).reshape(1, G * L)
    wfcrow = jnp.tile(jnp.zeros((L,), jnp.float32).at[:R2].set(W_fc[:, 0]),
                      (G,)).reshape(1, G * L)
    bfcp = b_fc.reshape(1, 1)
    cntm = jnp.asarray(_CNT_BCAST)
    gsum = jnp.asarray(_GSUM)

    n_pad = n_acc - N
    p1, xroot1 = pl.pallas_call(
        functools.partial(_tc_proj_kernel, n_pad),
        out_shape=[jax.ShapeDtypeStruct((R, G * L), jnp.float32),
                   jax.ShapeDtypeStruct((R, G * L), jnp.float32)],
    )(x, w1)

    seg_sum = _make_seg_sum(n_acc, j_chunks, rps)
    acc1 = seg_sum(eip, p1.reshape(n_acc, L)).reshape(NC, R, G * L)

    p2, hroot2 = pl.pallas_call(
        _tc_mid_kernel,
        out_shape=[jax.ShapeDtypeStruct((R, G * L), jnp.float32),
                   jax.ShapeDtypeStruct((R, G * L), jnp.float32)],
    )(acc1, xroot1, b1row, w2a, w2b, cntm)

    acc2 = seg_sum(eip, p2.reshape(n_acc, L)).reshape(NC, R, G * L)

    out8 = pl.pallas_call(
        functools.partial(_tc_final_kernel, N),
        out_shape=jax.ShapeDtypeStruct((R, G), jnp.float32),
    )(acc2, hroot2, b2row, wfcrow, bfcp, cntm, gsum)

    return out8.T.reshape(n_acc, 1)[:N]
